# baseline (speedup 1.0000x reference)
import functools

import jax
import jax.numpy as jnp
from jax import lax
from jax.experimental import pallas as pl
from jax.experimental.pallas import tpu as pltpu
from jax.experimental.pallas import tpu_sc as plsc

B = 1024
N = 100000
GAMMA = 2.0

ROW_BLK = 64
NUM_ROW_BLKS = B // ROW_BLK

_NC = 2
_NS = 16
_L = 16
_NW = _NC * _NS
_BPW = B // _NW


def _sc_gather(x_flat, targets, a_flat):
    mesh = plsc.VectorSubcoreMesh(core_axis_name="c", subcore_axis_name="s")

    @functools.partial(
        pl.kernel,
        mesh=mesh,
        out_type=(
            jax.ShapeDtypeStruct((B,), jnp.float32),
            jax.ShapeDtypeStruct((B,), jnp.float32),
        ),
        scratch_types=[
            pltpu.VMEM((_BPW,), jnp.int32),
            pltpu.VMEM((_BPW,), jnp.int32),
            pltpu.VMEM((_BPW,), jnp.float32),
            pltpu.VMEM((_BPW,), jnp.float32),
            pltpu.SemaphoreType.DMA,
        ],
    )
    def k(x_hbm, t_hbm, a_hbm, xt_hbm, at_hbm, tgt_v, idx_v, xt_v, at_v, sem):
        wid = lax.axis_index("s") * _NC + lax.axis_index("c")
        base = wid * _BPW
        pltpu.sync_copy(t_hbm.at[pl.ds(base, _BPW)], tgt_v)
        for j in range(_BPW // _L):
            t = tgt_v[pl.ds(j * _L, _L)]
            rows = base + j * _L + lax.iota(jnp.int32, _L)
            idx_v[pl.ds(j * _L, _L)] = rows * N + t
        pltpu.async_copy(x_hbm.at[idx_v], xt_v, sem).wait()
        pltpu.async_copy(a_hbm.at[tgt_v], at_v, sem).wait()
        pltpu.sync_copy(xt_v, xt_hbm.at[pl.ds(base, _BPW)])
        pltpu.sync_copy(at_v, at_hbm.at[pl.ds(base, _BPW)])

    return k(x_flat, targets, a_flat)


def _tc_loss_body(x_ref, xt_ref, at_ref, out_ref):
    r = pl.program_id(0)

    @pl.when(r == 0)
    def _init():
        out_ref[...] = jnp.zeros((1, 1), jnp.float32)

    x = x_ref[...]
    m = jnp.max(x, axis=1, keepdims=True)
    s = jnp.sum(jnp.exp(x - m), axis=1, keepdims=True)
    log_p = (xt_ref[...] - m) - jnp.log(s)
    one_m_p = 1.0 - jnp.exp(log_p)
    row_loss = -at_ref[...] * one_m_p * one_m_p * log_p
    out_ref[...] += (jnp.sum(row_loss) / B).reshape(1, 1)


def _tc_loss(inputs, xt, at):
    return pl.pallas_call(
        _tc_loss_body,
        grid=(NUM_ROW_BLKS,),
        in_specs=[
            pl.BlockSpec((ROW_BLK, N), lambda r: (r, 0)),
            pl.BlockSpec((ROW_BLK, 1), lambda r: (r, 0)),
            pl.BlockSpec((ROW_BLK, 1), lambda r: (r, 0)),
        ],
        out_specs=pl.BlockSpec((1, 1), lambda r: (0, 0)),
        out_shape=jax.ShapeDtypeStruct((1, 1), jnp.float32),
    )(inputs, xt, at)


def _loss_body(x_ref, xt_ref, at_ref, out_ref):
    r = pl.program_id(0)

    @pl.when(r == 0)
    def _init():
        out_ref[...] = jnp.zeros((1, 1), jnp.float32)

    x = x_ref[...]
    m = jnp.max(x, axis=1, keepdims=True)
    s = jnp.sum(jnp.exp(x - m), axis=1, keepdims=True)
    xt = xt_ref[pl.ds(r * ROW_BLK, ROW_BLK), :]
    at = at_ref[pl.ds(r * ROW_BLK, ROW_BLK), :]
    log_p = (xt - m) - jnp.log(s)
    one_m_p = 1.0 - jnp.exp(log_p)
    row_loss = -at * one_m_p * one_m_p * log_p
    out_ref[...] += (jnp.sum(row_loss) / B).reshape(1, 1)


def kernel(inputs, targets, alpha):
    targets = targets.reshape(-1).astype(jnp.int32)
    xt, at = _sc_gather(inputs.reshape(-1), targets, alpha.reshape(-1))
    loss = pl.pallas_call(
        _loss_body,
        grid=(NUM_ROW_BLKS,),
        in_specs=[
            pl.BlockSpec((ROW_BLK, N), lambda r: (r, 0)),
            pl.BlockSpec((B, 1), lambda r: (0, 0)),
            pl.BlockSpec((B, 1), lambda r: (0, 0)),
        ],
        out_specs=pl.BlockSpec((1, 1), lambda r: (0, 0)),
        out_shape=jax.ShapeDtypeStruct((1, 1), jnp.float32),
    )(inputs, xt.reshape(B, 1), at.reshape(B, 1))
    return loss[0, 0]


# speedup vs baseline: 1.9725x; 1.9725x over previous
import jax
import jax.numpy as jnp
from jax import lax
from jax.experimental import pallas as pl
from jax.experimental.pallas import tpu as pltpu

B = 1024
N = 100000
GAMMA = 2.0

ROW_BLK = 64
NUM_ROW_BLKS = B // ROW_BLK


def _loss_body(x_ref, t_ref, a_ref, out_ref):
    r = pl.program_id(0)

    @pl.when(r == 0)
    def _init():
        out_ref[...] = jnp.zeros((1, 1), jnp.float32)

    x = x_ref[...]
    m = jnp.max(x, axis=1, keepdims=True)
    t = t_ref[pl.ds(r * ROW_BLK, ROW_BLK), :]
    cols = lax.broadcasted_iota(jnp.int32, (ROW_BLK, N), 1)
    mask = cols == t
    e = jnp.exp(x - m)
    s = jnp.sum(e, axis=1, keepdims=True)
    xt = jnp.sum(jnp.where(mask, x, 0.0), axis=1, keepdims=True)
    at = jnp.sum(jnp.where(mask, a_ref[...], 0.0), axis=1, keepdims=True)
    log_p = (xt - m) - jnp.log(s)
    one_m_p = 1.0 - jnp.exp(log_p)
    row_loss = -at * one_m_p * one_m_p * log_p
    out_ref[...] += (jnp.sum(row_loss) / B).reshape(1, 1)


def kernel(inputs, targets, alpha):
    targets = targets.reshape(B, 1).astype(jnp.int32)
    alpha_row = alpha.reshape(1, N)
    loss = pl.pallas_call(
        _loss_body,
        grid=(NUM_ROW_BLKS,),
        in_specs=[
            pl.BlockSpec((ROW_BLK, N), lambda r: (r, 0)),
            pl.BlockSpec((B, 1), lambda r: (0, 0)),
            pl.BlockSpec((1, N), lambda r: (0, 0)),
        ],
        out_specs=pl.BlockSpec((1, 1), lambda r: (0, 0)),
        out_shape=jax.ShapeDtypeStruct((1, 1), jnp.float32),
    )(inputs, targets, alpha_row)
    return loss[0, 0]
